# Initial kernel scaffold; baseline (speedup 1.0000x reference)
import jax
import jax.numpy as jnp
from jax.experimental import pallas as pl


def kernel(points_position, sampled_points_position, skip_points_feature, sampled_points_feature, W0, b0, gamma0, beta0, W1, b1, gamma1, beta1):
    raise NotImplementedError("write your pallas kernel here")


# speedup vs baseline: 18.6478x; 18.6478x over previous
import functools

import jax
import jax.numpy as jnp
from jax.experimental import pallas as pl

B, N, S = 16, 4096, 1024
C = 3
D_SKIP, D_SAMP = 256, 512
BN = 512


def _k1(pp_ref, spp_ref, sfeat_ref, skip_ref, w0a_ref, w0b_ref, b0_ref,
        h1_ref, s_ref, sq_ref):
    pos = pp_ref[0]
    spos = spp_ref[0]
    mm = jax.lax.dot_general(pos, spos, (((0,), (0,)), ((), ())),
                             preferred_element_type=jnp.float32)
    pn = jnp.sum(pos * pos, axis=0)[:, None]
    sn = jnp.sum(spos * spos, axis=0)[None, :]
    d = -2.0 * mm + pn + sn

    iota = jax.lax.broadcasted_iota(jnp.int32, d.shape, 1)
    d_cur = d
    masks = []
    dmins = []
    for _ in range(3):
        a = jnp.argmin(d_cur, axis=1)
        m = jnp.min(d_cur, axis=1)
        mask = iota == a[:, None]
        masks.append(mask)
        dmins.append(m[:, None])
        d_cur = jnp.where(mask, jnp.inf, d_cur)
    wr = [1.0 / (m + 1e-8) for m in dmins]
    tot = wr[0] + wr[1] + wr[2]
    A = sum(jnp.where(masks[k], wr[k] / tot, 0.0) for k in range(3))

    interp = jax.lax.dot_general(A, sfeat_ref[0], (((1,), (1,)), ((), ())),
                                 preferred_element_type=jnp.float32)
    h = jax.lax.dot_general(skip_ref[0], w0a_ref[...],
                            (((0,), (1,)), ((), ())),
                            preferred_element_type=jnp.float32)
    h = h + jax.lax.dot_general(interp, w0b_ref[...],
                                (((1,), (1,)), ((), ())),
                                preferred_element_type=jnp.float32)
    h = h + b0_ref[...]
    h1_ref[0] = h

    step = pl.program_id(0) * pl.num_programs(1) + pl.program_id(1)

    @pl.when(step == 0)
    def _():
        s_ref[...] = jnp.zeros_like(s_ref)
        sq_ref[...] = jnp.zeros_like(sq_ref)

    s_ref[...] += jnp.sum(h, axis=0, keepdims=True)
    sq_ref[...] += jnp.sum(h * h, axis=0, keepdims=True)


def _k2(h1_ref, sc_ref, sh_ref, w1_ref, b1_ref, h2_ref, s_ref, sq_ref):
    h = jnp.maximum(h1_ref[0] * sc_ref[...] + sh_ref[...], 0.0)
    h2 = jax.lax.dot_general(h, w1_ref[...], (((1,), (1,)), ((), ())),
                             preferred_element_type=jnp.float32) + b1_ref[...]
    h2_ref[0] = h2

    step = pl.program_id(0) * pl.num_programs(1) + pl.program_id(1)

    @pl.when(step == 0)
    def _():
        s_ref[...] = jnp.zeros_like(s_ref)
        sq_ref[...] = jnp.zeros_like(sq_ref)

    s_ref[...] += jnp.sum(h2, axis=0, keepdims=True)
    sq_ref[...] += jnp.sum(h2 * h2, axis=0, keepdims=True)


def _k3(h2_ref, sc_ref, sh_ref, out_ref):
    h = jnp.maximum(h2_ref[0] * sc_ref[...] + sh_ref[...], 0.0)
    out_ref[0] = h.T


def _scale_shift(s, sq, gamma, beta):
    cnt = float(B * N)
    mean = s[0] / cnt
    var = sq[0] / cnt - mean * mean
    scale = gamma / jnp.sqrt(var + 1e-5)
    shift = beta - mean * scale
    return scale[None, :], shift[None, :]


@functools.partial(jax.jit, static_argnames=())
def kernel(points_position, sampled_points_position, skip_points_feature,
           sampled_points_feature, W0, b0, gamma0, beta0, W1, b1, gamma1,
           beta1):
    f32 = jnp.float32
    nb = N // BN
    w0a = W0[:, :D_SKIP]
    w0b = W0[:, D_SKIP:]

    h1, s1, sq1 = pl.pallas_call(
        _k1,
        grid=(B, nb),
        in_specs=[
            pl.BlockSpec((1, C, BN), lambda b, n: (b, 0, n)),
            pl.BlockSpec((1, C, S), lambda b, n: (b, 0, 0)),
            pl.BlockSpec((1, D_SAMP, S), lambda b, n: (b, 0, 0)),
            pl.BlockSpec((1, D_SKIP, BN), lambda b, n: (b, 0, n)),
            pl.BlockSpec((256, 256), lambda b, n: (0, 0)),
            pl.BlockSpec((256, D_SAMP), lambda b, n: (0, 0)),
            pl.BlockSpec((1, 256), lambda b, n: (0, 0)),
        ],
        out_specs=[
            pl.BlockSpec((1, BN, 256), lambda b, n: (b, n, 0)),
            pl.BlockSpec((1, 256), lambda b, n: (0, 0)),
            pl.BlockSpec((1, 256), lambda b, n: (0, 0)),
        ],
        out_shape=[
            jax.ShapeDtypeStruct((B, N, 256), f32),
            jax.ShapeDtypeStruct((1, 256), f32),
            jax.ShapeDtypeStruct((1, 256), f32),
        ],
    )(points_position, sampled_points_position, sampled_points_feature,
      skip_points_feature, w0a, w0b, b0[None, :])

    sc1, sh1 = _scale_shift(s1, sq1, gamma0, beta0)

    h2, s2, sq2 = pl.pallas_call(
        _k2,
        grid=(B, nb),
        in_specs=[
            pl.BlockSpec((1, BN, 256), lambda b, n: (b, n, 0)),
            pl.BlockSpec((1, 256), lambda b, n: (0, 0)),
            pl.BlockSpec((1, 256), lambda b, n: (0, 0)),
            pl.BlockSpec((256, 256), lambda b, n: (0, 0)),
            pl.BlockSpec((1, 256), lambda b, n: (0, 0)),
        ],
        out_specs=[
            pl.BlockSpec((1, BN, 256), lambda b, n: (b, n, 0)),
            pl.BlockSpec((1, 256), lambda b, n: (0, 0)),
            pl.BlockSpec((1, 256), lambda b, n: (0, 0)),
        ],
        out_shape=[
            jax.ShapeDtypeStruct((B, N, 256), f32),
            jax.ShapeDtypeStruct((1, 256), f32),
            jax.ShapeDtypeStruct((1, 256), f32),
        ],
    )(h1, sc1, sh1, W1, b1[None, :])

    sc2, sh2 = _scale_shift(s2, sq2, gamma1, beta1)

    out = pl.pallas_call(
        _k3,
        grid=(B, nb),
        in_specs=[
            pl.BlockSpec((1, BN, 256), lambda b, n: (b, n, 0)),
            pl.BlockSpec((1, 256), lambda b, n: (0, 0)),
            pl.BlockSpec((1, 256), lambda b, n: (0, 0)),
        ],
        out_specs=pl.BlockSpec((1, 256, BN), lambda b, n: (b, 0, n)),
        out_shape=jax.ShapeDtypeStruct((B, 256, N), f32),
    )(h2, sc2, sh2)

    return out


# speedup vs baseline: 22.4512x; 1.2040x over previous
import functools

import jax
import jax.numpy as jnp
from jax.experimental import pallas as pl

B, N, S = 16, 4096, 1024
C = 3
D_SKIP, D_SAMP = 256, 512
BN = 512


def _k1(pp_ref, spp_ref, sfeat_ref, skip_ref, w0a_ref, w0b_ref, b0_ref,
        h1_ref, s_ref, sq_ref):
    pos = pp_ref[0]
    spos = spp_ref[0]
    mm = jax.lax.dot_general(pos, spos, (((0,), (0,)), ((), ())),
                             preferred_element_type=jnp.float32)
    pn = jnp.sum(pos * pos, axis=0)[:, None]
    sn = jnp.sum(spos * spos, axis=0)[None, :]
    d = -2.0 * mm + pn + sn

    iota = jax.lax.broadcasted_iota(jnp.int32, d.shape, 1)
    d_cur = d
    A = jnp.zeros_like(d)
    tot = jnp.zeros((d.shape[0], 1), jnp.float32)
    for k in range(3):
        m = jnp.min(d_cur, axis=1, keepdims=True)
        t = jnp.where(d_cur == m, iota, S)
        i0 = jnp.min(t, axis=1, keepdims=True)
        wr = 1.0 / (m + 1e-8)
        tot = tot + wr
        sel = iota == i0
        A = A + jnp.where(sel, wr, 0.0)
        if k < 2:
            d_cur = jnp.where(sel, jnp.inf, d_cur)

    interp = jax.lax.dot_general(A, sfeat_ref[0], (((1,), (1,)), ((), ())),
                                 preferred_element_type=jnp.float32)
    interp = interp * (1.0 / tot)
    h = jax.lax.dot_general(skip_ref[0], w0a_ref[...],
                            (((0,), (1,)), ((), ())),
                            preferred_element_type=jnp.float32)
    h = h + jax.lax.dot_general(interp, w0b_ref[...],
                                (((1,), (1,)), ((), ())),
                                preferred_element_type=jnp.float32)
    h = h + b0_ref[...]
    h1_ref[0] = h

    step = pl.program_id(0) * pl.num_programs(1) + pl.program_id(1)

    @pl.when(step == 0)
    def _():
        s_ref[...] = jnp.zeros_like(s_ref)
        sq_ref[...] = jnp.zeros_like(sq_ref)

    s_ref[...] += jnp.sum(h, axis=0, keepdims=True)
    sq_ref[...] += jnp.sum(h * h, axis=0, keepdims=True)


def _k2(h1_ref, sc_ref, sh_ref, w1_ref, b1_ref, h2_ref, s_ref, sq_ref):
    h = jnp.maximum(h1_ref[0] * sc_ref[...] + sh_ref[...], 0.0)
    h2 = jax.lax.dot_general(h, w1_ref[...], (((1,), (1,)), ((), ())),
                             preferred_element_type=jnp.float32) + b1_ref[...]
    h2_ref[0] = h2

    step = pl.program_id(0) * pl.num_programs(1) + pl.program_id(1)

    @pl.when(step == 0)
    def _():
        s_ref[...] = jnp.zeros_like(s_ref)
        sq_ref[...] = jnp.zeros_like(sq_ref)

    s_ref[...] += jnp.sum(h2, axis=0, keepdims=True)
    sq_ref[...] += jnp.sum(h2 * h2, axis=0, keepdims=True)


def _k3(h2_ref, sc_ref, sh_ref, out_ref):
    h = jnp.maximum(h2_ref[0] * sc_ref[...] + sh_ref[...], 0.0)
    out_ref[0] = h.T


def _scale_shift(s, sq, gamma, beta):
    cnt = float(B * N)
    mean = s[0] / cnt
    var = sq[0] / cnt - mean * mean
    scale = gamma / jnp.sqrt(var + 1e-5)
    shift = beta - mean * scale
    return scale[None, :], shift[None, :]


@functools.partial(jax.jit, static_argnames=())
def kernel(points_position, sampled_points_position, skip_points_feature,
           sampled_points_feature, W0, b0, gamma0, beta0, W1, b1, gamma1,
           beta1):
    f32 = jnp.float32
    nb = N // BN
    w0a = W0[:, :D_SKIP]
    w0b = W0[:, D_SKIP:]

    h1, s1, sq1 = pl.pallas_call(
        _k1,
        grid=(B, nb),
        in_specs=[
            pl.BlockSpec((1, C, BN), lambda b, n: (b, 0, n)),
            pl.BlockSpec((1, C, S), lambda b, n: (b, 0, 0)),
            pl.BlockSpec((1, D_SAMP, S), lambda b, n: (b, 0, 0)),
            pl.BlockSpec((1, D_SKIP, BN), lambda b, n: (b, 0, n)),
            pl.BlockSpec((256, 256), lambda b, n: (0, 0)),
            pl.BlockSpec((256, D_SAMP), lambda b, n: (0, 0)),
            pl.BlockSpec((1, 256), lambda b, n: (0, 0)),
        ],
        out_specs=[
            pl.BlockSpec((1, BN, 256), lambda b, n: (b, n, 0)),
            pl.BlockSpec((1, 256), lambda b, n: (0, 0)),
            pl.BlockSpec((1, 256), lambda b, n: (0, 0)),
        ],
        out_shape=[
            jax.ShapeDtypeStruct((B, N, 256), f32),
            jax.ShapeDtypeStruct((1, 256), f32),
            jax.ShapeDtypeStruct((1, 256), f32),
        ],
    )(points_position, sampled_points_position, sampled_points_feature,
      skip_points_feature, w0a, w0b, b0[None, :])

    sc1, sh1 = _scale_shift(s1, sq1, gamma0, beta0)

    h2, s2, sq2 = pl.pallas_call(
        _k2,
        grid=(B, nb),
        in_specs=[
            pl.BlockSpec((1, BN, 256), lambda b, n: (b, n, 0)),
            pl.BlockSpec((1, 256), lambda b, n: (0, 0)),
            pl.BlockSpec((1, 256), lambda b, n: (0, 0)),
            pl.BlockSpec((256, 256), lambda b, n: (0, 0)),
            pl.BlockSpec((1, 256), lambda b, n: (0, 0)),
        ],
        out_specs=[
            pl.BlockSpec((1, BN, 256), lambda b, n: (b, n, 0)),
            pl.BlockSpec((1, 256), lambda b, n: (0, 0)),
            pl.BlockSpec((1, 256), lambda b, n: (0, 0)),
        ],
        out_shape=[
            jax.ShapeDtypeStruct((B, N, 256), f32),
            jax.ShapeDtypeStruct((1, 256), f32),
            jax.ShapeDtypeStruct((1, 256), f32),
        ],
    )(h1, sc1, sh1, W1, b1[None, :])

    sc2, sh2 = _scale_shift(s2, sq2, gamma1, beta1)

    out = pl.pallas_call(
        _k3,
        grid=(B, nb),
        in_specs=[
            pl.BlockSpec((1, BN, 256), lambda b, n: (b, n, 0)),
            pl.BlockSpec((1, 256), lambda b, n: (0, 0)),
            pl.BlockSpec((1, 256), lambda b, n: (0, 0)),
        ],
        out_specs=pl.BlockSpec((1, 256, BN), lambda b, n: (b, 0, n)),
        out_shape=jax.ShapeDtypeStruct((B, 256, N), f32),
    )(h2, sc2, sh2)

    return out


# speedup vs baseline: 25.5623x; 1.1386x over previous
import functools

import jax
import jax.numpy as jnp
from jax.experimental import pallas as pl

B, N, S = 16, 4096, 1024
C = 3
D_SKIP, D_SAMP = 256, 512
BN = 1024


def _k1(pp_ref, spp_ref, sfeat_ref, skip_ref, w0a_ref, w0b_ref, b0_ref,
        h1_ref, s_ref, sq_ref):
    pos = pp_ref[0]
    spos = spp_ref[0]
    mm = jax.lax.dot_general(pos, spos, (((0,), (0,)), ((), ())),
                             preferred_element_type=jnp.float32)
    pn = jnp.sum(pos * pos, axis=0)[:, None]
    sn = jnp.sum(spos * spos, axis=0)[None, :]
    d = -2.0 * mm + pn + sn

    iota = jax.lax.broadcasted_iota(jnp.int32, d.shape, 1)
    d_cur = d
    A = jnp.zeros_like(d)
    tot = jnp.zeros((d.shape[0], 1), jnp.float32)
    for k in range(3):
        m = jnp.min(d_cur, axis=1, keepdims=True)
        t = jnp.where(d_cur == m, iota, S)
        i0 = jnp.min(t, axis=1, keepdims=True)
        wr = 1.0 / (m + 1e-8)
        tot = tot + wr
        sel = iota == i0
        A = A + jnp.where(sel, wr, 0.0)
        if k < 2:
            d_cur = jnp.where(sel, jnp.inf, d_cur)

    interp = jax.lax.dot_general(A, sfeat_ref[0], (((1,), (1,)), ((), ())),
                                 preferred_element_type=jnp.float32)
    interp = interp * (1.0 / tot)
    h = jax.lax.dot_general(skip_ref[0], w0a_ref[...],
                            (((0,), (1,)), ((), ())),
                            preferred_element_type=jnp.float32)
    h = h + jax.lax.dot_general(interp, w0b_ref[...],
                                (((1,), (1,)), ((), ())),
                                preferred_element_type=jnp.float32)
    h = h + b0_ref[...]
    h1_ref[0] = h

    step = pl.program_id(0) * pl.num_programs(1) + pl.program_id(1)

    @pl.when(step == 0)
    def _():
        s_ref[...] = jnp.zeros_like(s_ref)
        sq_ref[...] = jnp.zeros_like(sq_ref)

    s_ref[...] += jnp.sum(h, axis=0, keepdims=True)
    sq_ref[...] += jnp.sum(h * h, axis=0, keepdims=True)


def _k2(h1_ref, sc_ref, sh_ref, w1_ref, b1_ref, h2_ref, s_ref, sq_ref):
    h = jnp.maximum(h1_ref[0] * sc_ref[...] + sh_ref[...], 0.0)
    h2 = jax.lax.dot_general(h, w1_ref[...], (((1,), (1,)), ((), ())),
                             preferred_element_type=jnp.float32) + b1_ref[...]
    h2_ref[0] = h2

    step = pl.program_id(0) * pl.num_programs(1) + pl.program_id(1)

    @pl.when(step == 0)
    def _():
        s_ref[...] = jnp.zeros_like(s_ref)
        sq_ref[...] = jnp.zeros_like(sq_ref)

    s_ref[...] += jnp.sum(h2, axis=0, keepdims=True)
    sq_ref[...] += jnp.sum(h2 * h2, axis=0, keepdims=True)


def _k3(h2_ref, sc_ref, sh_ref, out_ref):
    h = jnp.maximum(h2_ref[0] * sc_ref[...] + sh_ref[...], 0.0)
    out_ref[0] = h.T


def _scale_shift(s, sq, gamma, beta):
    cnt = float(B * N)
    mean = s[0] / cnt
    var = sq[0] / cnt - mean * mean
    scale = gamma / jnp.sqrt(var + 1e-5)
    shift = beta - mean * scale
    return scale[None, :], shift[None, :]


@functools.partial(jax.jit, static_argnames=())
def kernel(points_position, sampled_points_position, skip_points_feature,
           sampled_points_feature, W0, b0, gamma0, beta0, W1, b1, gamma1,
           beta1):
    f32 = jnp.float32
    nb = N // BN
    w0a = W0[:, :D_SKIP]
    w0b = W0[:, D_SKIP:]

    h1, s1, sq1 = pl.pallas_call(
        _k1,
        grid=(B, nb),
        in_specs=[
            pl.BlockSpec((1, C, BN), lambda b, n: (b, 0, n)),
            pl.BlockSpec((1, C, S), lambda b, n: (b, 0, 0)),
            pl.BlockSpec((1, D_SAMP, S), lambda b, n: (b, 0, 0)),
            pl.BlockSpec((1, D_SKIP, BN), lambda b, n: (b, 0, n)),
            pl.BlockSpec((256, 256), lambda b, n: (0, 0)),
            pl.BlockSpec((256, D_SAMP), lambda b, n: (0, 0)),
            pl.BlockSpec((1, 256), lambda b, n: (0, 0)),
        ],
        out_specs=[
            pl.BlockSpec((1, BN, 256), lambda b, n: (b, n, 0)),
            pl.BlockSpec((1, 256), lambda b, n: (0, 0)),
            pl.BlockSpec((1, 256), lambda b, n: (0, 0)),
        ],
        out_shape=[
            jax.ShapeDtypeStruct((B, N, 256), f32),
            jax.ShapeDtypeStruct((1, 256), f32),
            jax.ShapeDtypeStruct((1, 256), f32),
        ],
    )(points_position, sampled_points_position, sampled_points_feature,
      skip_points_feature, w0a, w0b, b0[None, :])

    sc1, sh1 = _scale_shift(s1, sq1, gamma0, beta0)

    h2, s2, sq2 = pl.pallas_call(
        _k2,
        grid=(B, nb),
        in_specs=[
            pl.BlockSpec((1, BN, 256), lambda b, n: (b, n, 0)),
            pl.BlockSpec((1, 256), lambda b, n: (0, 0)),
            pl.BlockSpec((1, 256), lambda b, n: (0, 0)),
            pl.BlockSpec((256, 256), lambda b, n: (0, 0)),
            pl.BlockSpec((1, 256), lambda b, n: (0, 0)),
        ],
        out_specs=[
            pl.BlockSpec((1, BN, 256), lambda b, n: (b, n, 0)),
            pl.BlockSpec((1, 256), lambda b, n: (0, 0)),
            pl.BlockSpec((1, 256), lambda b, n: (0, 0)),
        ],
        out_shape=[
            jax.ShapeDtypeStruct((B, N, 256), f32),
            jax.ShapeDtypeStruct((1, 256), f32),
            jax.ShapeDtypeStruct((1, 256), f32),
        ],
    )(h1, sc1, sh1, W1, b1[None, :])

    sc2, sh2 = _scale_shift(s2, sq2, gamma1, beta1)

    out = pl.pallas_call(
        _k3,
        grid=(B, nb),
        in_specs=[
            pl.BlockSpec((1, BN, 256), lambda b, n: (b, n, 0)),
            pl.BlockSpec((1, 256), lambda b, n: (0, 0)),
            pl.BlockSpec((1, 256), lambda b, n: (0, 0)),
        ],
        out_specs=pl.BlockSpec((1, 256, BN), lambda b, n: (b, 0, n)),
        out_shape=jax.ShapeDtypeStruct((B, 256, N), f32),
    )(h2, sc2, sh2)

    return out


# speedup vs baseline: 27.5900x; 1.0793x over previous
import functools

import jax
import jax.numpy as jnp
from jax.experimental import pallas as pl

B, N, S = 16, 4096, 1024
C = 3
D_SKIP, D_SAMP = 256, 512
BN = 1024


def _k1(pp_ref, spp_ref, sfeat_ref, skip_ref, w0a_ref, w0b_ref, b0_ref,
        h1_ref, s_ref, sq_ref):
    pos = pp_ref[0]
    spos = spp_ref[0]
    mm = jax.lax.dot_general(pos, spos, (((0,), (0,)), ((), ())),
                             preferred_element_type=jnp.float32)
    pn = jnp.sum(pos * pos, axis=0)[:, None]
    sn = jnp.sum(spos * spos, axis=0)[None, :]
    d = -2.0 * mm + pn + sn

    iota = jax.lax.broadcasted_iota(jnp.int32, d.shape, 1).astype(jnp.float32)
    d_cur = d
    A = jnp.zeros_like(d)
    tot = jnp.zeros((d.shape[0], 1), jnp.float32)
    for k in range(3):
        m = jnp.min(d_cur, axis=1, keepdims=True)
        t = jnp.where(d_cur == m, iota, float(S))
        i0 = jnp.min(t, axis=1, keepdims=True)
        wr = 1.0 / (m + 1e-8)
        tot = tot + wr
        sel = t == i0
        A = A + jnp.where(sel, wr, 0.0)
        if k < 2:
            d_cur = jnp.where(sel, jnp.inf, d_cur)

    interp = jax.lax.dot_general(A, sfeat_ref[0], (((1,), (1,)), ((), ())),
                                 preferred_element_type=jnp.float32)
    interp = interp * (1.0 / tot)
    h = jax.lax.dot_general(skip_ref[0], w0a_ref[...],
                            (((0,), (1,)), ((), ())),
                            preferred_element_type=jnp.float32)
    h = h + jax.lax.dot_general(interp, w0b_ref[...],
                                (((1,), (1,)), ((), ())),
                                preferred_element_type=jnp.float32)
    h = h + b0_ref[...]
    h1_ref[0] = h

    step = pl.program_id(0) * pl.num_programs(1) + pl.program_id(1)

    @pl.when(step == 0)
    def _():
        s_ref[...] = jnp.zeros_like(s_ref)
        sq_ref[...] = jnp.zeros_like(sq_ref)

    s_ref[...] += jnp.sum(h, axis=0, keepdims=True)
    sq_ref[...] += jnp.sum(h * h, axis=0, keepdims=True)


def _k2(h1_ref, sc_ref, sh_ref, w1_ref, b1_ref, h2_ref, s_ref, sq_ref):
    h = jnp.maximum(h1_ref[0] * sc_ref[...] + sh_ref[...], 0.0)
    h2 = jax.lax.dot_general(h, w1_ref[...], (((1,), (1,)), ((), ())),
                             preferred_element_type=jnp.float32) + b1_ref[...]
    h2_ref[0] = h2

    step = pl.program_id(0) * pl.num_programs(1) + pl.program_id(1)

    @pl.when(step == 0)
    def _():
        s_ref[...] = jnp.zeros_like(s_ref)
        sq_ref[...] = jnp.zeros_like(sq_ref)

    s_ref[...] += jnp.sum(h2, axis=0, keepdims=True)
    sq_ref[...] += jnp.sum(h2 * h2, axis=0, keepdims=True)


def _k3(h2_ref, sc_ref, sh_ref, out_ref):
    h = jnp.maximum(h2_ref[0] * sc_ref[...] + sh_ref[...], 0.0)
    out_ref[0] = h.T


def _scale_shift(s, sq, gamma, beta):
    cnt = float(B * N)
    mean = s[0] / cnt
    var = sq[0] / cnt - mean * mean
    scale = gamma / jnp.sqrt(var + 1e-5)
    shift = beta - mean * scale
    return scale[None, :], shift[None, :]


@functools.partial(jax.jit, static_argnames=())
def kernel(points_position, sampled_points_position, skip_points_feature,
           sampled_points_feature, W0, b0, gamma0, beta0, W1, b1, gamma1,
           beta1):
    f32 = jnp.float32
    nb = N // BN
    w0a = W0[:, :D_SKIP]
    w0b = W0[:, D_SKIP:]

    h1, s1, sq1 = pl.pallas_call(
        _k1,
        grid=(B, nb),
        in_specs=[
            pl.BlockSpec((1, C, BN), lambda b, n: (b, 0, n)),
            pl.BlockSpec((1, C, S), lambda b, n: (b, 0, 0)),
            pl.BlockSpec((1, D_SAMP, S), lambda b, n: (b, 0, 0)),
            pl.BlockSpec((1, D_SKIP, BN), lambda b, n: (b, 0, n)),
            pl.BlockSpec((256, 256), lambda b, n: (0, 0)),
            pl.BlockSpec((256, D_SAMP), lambda b, n: (0, 0)),
            pl.BlockSpec((1, 256), lambda b, n: (0, 0)),
        ],
        out_specs=[
            pl.BlockSpec((1, BN, 256), lambda b, n: (b, n, 0)),
            pl.BlockSpec((1, 256), lambda b, n: (0, 0)),
            pl.BlockSpec((1, 256), lambda b, n: (0, 0)),
        ],
        out_shape=[
            jax.ShapeDtypeStruct((B, N, 256), f32),
            jax.ShapeDtypeStruct((1, 256), f32),
            jax.ShapeDtypeStruct((1, 256), f32),
        ],
    )(points_position, sampled_points_position, sampled_points_feature,
      skip_points_feature, w0a, w0b, b0[None, :])

    sc1, sh1 = _scale_shift(s1, sq1, gamma0, beta0)

    h2, s2, sq2 = pl.pallas_call(
        _k2,
        grid=(B, nb),
        in_specs=[
            pl.BlockSpec((1, BN, 256), lambda b, n: (b, n, 0)),
            pl.BlockSpec((1, 256), lambda b, n: (0, 0)),
            pl.BlockSpec((1, 256), lambda b, n: (0, 0)),
            pl.BlockSpec((256, 256), lambda b, n: (0, 0)),
            pl.BlockSpec((1, 256), lambda b, n: (0, 0)),
        ],
        out_specs=[
            pl.BlockSpec((1, BN, 256), lambda b, n: (b, n, 0)),
            pl.BlockSpec((1, 256), lambda b, n: (0, 0)),
            pl.BlockSpec((1, 256), lambda b, n: (0, 0)),
        ],
        out_shape=[
            jax.ShapeDtypeStruct((B, N, 256), f32),
            jax.ShapeDtypeStruct((1, 256), f32),
            jax.ShapeDtypeStruct((1, 256), f32),
        ],
    )(h1, sc1, sh1, W1, b1[None, :])

    sc2, sh2 = _scale_shift(s2, sq2, gamma1, beta1)

    out = pl.pallas_call(
        _k3,
        grid=(B, nb),
        in_specs=[
            pl.BlockSpec((1, BN, 256), lambda b, n: (b, n, 0)),
            pl.BlockSpec((1, 256), lambda b, n: (0, 0)),
            pl.BlockSpec((1, 256), lambda b, n: (0, 0)),
        ],
        out_specs=pl.BlockSpec((1, 256, BN), lambda b, n: (b, 0, n)),
        out_shape=jax.ShapeDtypeStruct((B, 256, N), f32),
    )(h2, sc2, sh2)

    return out


# speedup vs baseline: 31.2899x; 1.1341x over previous
import functools

import jax
import jax.numpy as jnp
from jax.experimental import pallas as pl
from jax.experimental.pallas import tpu as pltpu

B, N, S = 16, 4096, 1024
C = 3
D_SKIP, D_SAMP = 256, 512
BN = 1024
NB = N // BN
CNT = 1.0 / float(B * N)


def _mega(pp_ref, spp_ref, sfeat_ref, skip_ref, w0a_ref, w0b_ref, b0_ref,
          w1_ref, b1_ref, g0_ref, be0_ref, g1_ref, be1_ref,
          out_ref, h_scr, s1, sq1, s2, sq2):
    p = pl.program_id(0)
    b = pl.program_id(1)
    n = pl.program_id(2)
    row0 = (b * NB + n) * BN

    @pl.when(p == 0)
    def _phase0():
        pos = pp_ref[0]
        spos = spp_ref[0]
        mm = jax.lax.dot_general(pos, spos, (((0,), (0,)), ((), ())),
                                 preferred_element_type=jnp.float32)
        pn = jnp.sum(pos * pos, axis=0)[:, None]
        sn = jnp.sum(spos * spos, axis=0)[None, :]
        d = -2.0 * mm + pn + sn

        iota = jax.lax.broadcasted_iota(jnp.int32, d.shape, 1).astype(
            jnp.float32)
        d_cur = d
        A = jnp.zeros_like(d)
        tot = jnp.zeros((d.shape[0], 1), jnp.float32)
        for k in range(3):
            m = jnp.min(d_cur, axis=1, keepdims=True)
            t = jnp.where(d_cur == m, iota, float(S))
            i0 = jnp.min(t, axis=1, keepdims=True)
            wr = 1.0 / (m + 1e-8)
            tot = tot + wr
            sel = t == i0
            A = A + jnp.where(sel, wr, 0.0)
            if k < 2:
                d_cur = jnp.where(sel, jnp.inf, d_cur)

        interp = jax.lax.dot_general(A, sfeat_ref[0], (((1,), (1,)), ((), ())),
                                     preferred_element_type=jnp.float32)
        interp = interp * (1.0 / tot)
        h = jax.lax.dot_general(skip_ref[0], w0a_ref[...],
                                (((0,), (1,)), ((), ())),
                                preferred_element_type=jnp.float32)
        h = h + jax.lax.dot_general(interp, w0b_ref[...],
                                    (((1,), (1,)), ((), ())),
                                    preferred_element_type=jnp.float32)
        h = h + b0_ref[...]
        h_scr[pl.ds(row0, BN), :] = h.astype(jnp.bfloat16)

        @pl.when((b == 0) & (n == 0))
        def _():
            s1[...] = jnp.zeros_like(s1)
            sq1[...] = jnp.zeros_like(sq1)

        s1[...] += jnp.sum(h, axis=0, keepdims=True)
        sq1[...] += jnp.sum(h * h, axis=0, keepdims=True)

    @pl.when(p == 1)
    def _phase1():
        mean = s1[...] * CNT
        var = sq1[...] * CNT - mean * mean
        sc = g0_ref[...] / jnp.sqrt(var + 1e-5)
        sh = be0_ref[...] - mean * sc
        h1 = jnp.maximum(
            h_scr[pl.ds(row0, BN), :].astype(jnp.float32) * sc + sh, 0.0)
        h2 = jax.lax.dot_general(h1, w1_ref[...], (((1,), (1,)), ((), ())),
                                 preferred_element_type=jnp.float32)
        h2 = h2 + b1_ref[...]
        h_scr[pl.ds(row0, BN), :] = h2.astype(jnp.bfloat16)

        @pl.when((b == 0) & (n == 0))
        def _():
            s2[...] = jnp.zeros_like(s2)
            sq2[...] = jnp.zeros_like(sq2)

        s2[...] += jnp.sum(h2, axis=0, keepdims=True)
        sq2[...] += jnp.sum(h2 * h2, axis=0, keepdims=True)

    @pl.when(p == 2)
    def _phase2():
        mean = s2[...] * CNT
        var = sq2[...] * CNT - mean * mean
        sc = g1_ref[...] / jnp.sqrt(var + 1e-5)
        sh = be1_ref[...] - mean * sc
        h = jnp.maximum(
            h_scr[pl.ds(row0, BN), :].astype(jnp.float32) * sc + sh, 0.0)
        out_ref[0] = h.T


def _c(shape):
    return pl.BlockSpec(shape, lambda p, b, n: tuple(0 for _ in shape))


@functools.partial(jax.jit, static_argnames=())
def kernel(points_position, sampled_points_position, skip_points_feature,
           sampled_points_feature, W0, b0, gamma0, beta0, W1, b1, gamma1,
           beta1):
    f32 = jnp.float32
    w0a = W0[:, :D_SKIP]
    w0b = W0[:, D_SKIP:]

    def bmap(p, b, n):
        return (jnp.where(p == 0, b, B - 1), 0, 0)

    def bnmap(p, b, n):
        return (jnp.where(p == 0, b, B - 1), 0, jnp.where(p == 0, n, NB - 1))

    out = pl.pallas_call(
        _mega,
        grid=(3, B, NB),
        in_specs=[
            pl.BlockSpec((1, C, BN), bnmap),
            pl.BlockSpec((1, C, S), bmap),
            pl.BlockSpec((1, D_SAMP, S), bmap),
            pl.BlockSpec((1, D_SKIP, BN), bnmap),
            _c((256, 256)),
            _c((256, D_SAMP)),
            _c((1, 256)),
            _c((256, 256)),
            _c((1, 256)),
            _c((1, 256)),
            _c((1, 256)),
            _c((1, 256)),
            _c((1, 256)),
        ],
        out_specs=pl.BlockSpec(
            (1, 256, BN),
            lambda p, b, n: (jnp.where(p == 2, b, 0), 0,
                             jnp.where(p == 2, n, 0))),
        out_shape=jax.ShapeDtypeStruct((B, 256, N), f32),
        scratch_shapes=[
            pltpu.VMEM((B * N, 256), jnp.bfloat16),
            pltpu.VMEM((1, 256), f32),
            pltpu.VMEM((1, 256), f32),
            pltpu.VMEM((1, 256), f32),
            pltpu.VMEM((1, 256), f32),
        ],
    )(points_position, sampled_points_position, sampled_points_feature,
      skip_points_feature, w0a, w0b, b0[None, :], W1, b1[None, :],
      gamma0[None, :], beta0[None, :], gamma1[None, :], beta1[None, :])

    return out
